# baseline (speedup 1.0000x reference)
import functools

import jax
import jax.numpy as jnp
from jax.experimental import pallas as pl
from jax.experimental.pallas import tpu as pltpu

_B = 16

_CONVS = ((32, (0, 30), False),
          (30, (0, 8, 14, 22, 28), True),
          (14, (0, 6, 12), False),
          (12, (0, 6, 10), True))


def _band_mats(w, Cout, j_lo, j_hi, pool):
    Cin = w.shape[0]
    OWp = j_hi - j_lo
    Wp = OWp + 2
    w4 = w.reshape(Cin, 3, 3, Cout)
    j = jnp.arange(OWp)
    mats = []
    for di in range(3):
        M = jnp.zeros((Wp, Cin, OWp, Cout), w.dtype)
        for dj in range(3):
            M = M.at[j + dj, :, j, :].add(w4[:, di, dj, :])
        if pool:
            M = M.reshape(Wp * Cin, OWp // 2, 2, Cout).transpose(0, 2, 1, 3)
        mats.append(M.reshape(Wp * Cin, OWp * Cout))
    return mats


def _bconv(x, pieces, brow, pool):
    B, H, WC = x.shape
    OH = H - 2
    outs = []
    for lane_lo, ms in pieces:
        K = ms[0].shape[0]
        OWC = ms[0].shape[1]
        xs = x[:, :, lane_lo:lane_lo + K]
        acc = jnp.dot(xs[:, 0:OH, :].reshape(B * OH, K), ms[0],
                      preferred_element_type=jnp.float32)
        acc = acc + jnp.dot(xs[:, 1:1 + OH, :].reshape(B * OH, K), ms[1],
                            preferred_element_type=jnp.float32)
        acc = acc + jnp.dot(xs[:, 2:2 + OH, :].reshape(B * OH, K), ms[2],
                            preferred_element_type=jnp.float32)
        o = jnp.maximum(acc + brow[:, :OWC], 0.0).astype(jnp.bfloat16)
        o = o.reshape(B, OH, OWC)
        if pool:
            v = o.reshape(B, OH // 2, 2, OWC)
            m = jnp.maximum(v[:, :, 0, :], v[:, :, 1, :])
            o = jnp.maximum(m[:, :, :OWC // 2], m[:, :, OWC // 2:])
        outs.append(o)
    return outs[0] if len(outs) == 1 else jnp.concatenate(outs, axis=2)


def _ffnet_kernel(cfg, x_ref, *refs):
    o_ref = refs[-1]
    vals = [r[...] for r in refs[:-1]]
    B = x_ref.shape[0]
    a = x_ref[...]
    idx = 0
    for lane_los, pool in cfg:
        pieces = []
        for lo in lane_los:
            pieces.append((lo, vals[idx:idx + 3]))
            idx += 3
        brow = vals[idx]
        idx += 1
        a = _bconv(a, pieces, brow, pool)
    f1w, f1b, f2w, f2b = vals[idx:idx + 4]
    h = jnp.dot(a.reshape(B, 1600), f1w, preferred_element_type=jnp.float32)
    h = jnp.maximum(h + f1b, 0.0).astype(jnp.bfloat16)
    y = jnp.dot(h, f2w, preferred_element_type=jnp.float32)
    o_ref[...] = y + f2b


@jax.jit
def _run(x_nchw, c1w, c1b, c2w, c2b, c3w, c3b, c4w, c4b,
         f1w, f1b, f2w, f2b):
    N = x_nchw.shape[0]
    B = _B
    while N % B:
        B //= 2
    x = jnp.transpose(x_nchw, (0, 2, 3, 1)).astype(jnp.bfloat16)
    x = x.reshape(N, 32, 32 * 3)

    ws = []
    cfg = []
    for (w, b), (W, splits, pool) in zip(
            ((c1w, c1b), (c2w, c2b), (c3w, c3b), (c4w, c4b)), _CONVS):
        Cin, Cout = w.shape[0], b.shape[1]
        for j_lo, j_hi in zip(splits[:-1], splits[1:]):
            ws += _band_mats(w, Cout, j_lo, j_hi, pool)
        ws.append(jnp.tile(b, (1, W - 2)))
        cfg.append((tuple(lo * Cin for lo in splits[:-1]), pool))
    ws += [f1w, f1b, f2w, f2b]

    def _full(w):
        return pl.BlockSpec(w.shape, lambda i, n=w.ndim: (0,) * n)

    out = pl.pallas_call(
        functools.partial(_ffnet_kernel, tuple(cfg)),
        out_shape=jax.ShapeDtypeStruct((N, 128), jnp.float32),
        grid=(N // B,),
        in_specs=[pl.BlockSpec((B, 32, 32 * 3), lambda i: (i, 0, 0))]
        + [_full(w) for w in ws],
        out_specs=pl.BlockSpec((B, 128), lambda i: (i, 0)),
        compiler_params=pltpu.CompilerParams(
            dimension_semantics=("parallel",)),
    )(x, *ws)
    return out[:, :10]


def kernel(x_nchw, conv1_w, conv1_b, conv2_w, conv2_b, conv3_w, conv3_b,
           conv4_w, conv4_b, fc1_w, fc1_b, fc2_w, fc2_b):
    return _run(x_nchw, conv1_w, conv1_b, conv2_w, conv2_b, conv3_w, conv3_b,
                conv4_w, conv4_b, fc1_w, fc1_b, fc2_w, fc2_b)


# speedup vs baseline: 1.3968x; 1.3968x over previous
import functools

import jax
import jax.numpy as jnp
from jax.experimental import pallas as pl
from jax.experimental.pallas import tpu as pltpu

_B = 32

_CONVS = ((32, (0, 30), False, 32),
          (30, (0, 8, 14, 22, 28), True, 16),
          (14, (0, 6, 12), False, 16),
          (12, (0, 6, 10), True, 5))


def _band_mats(w, Cout, j_lo, j_hi, pool):
    Cin = w.shape[0]
    OWp = j_hi - j_lo
    Wp = OWp + 2
    w4 = w.reshape(Cin, 3, 3, Cout)
    j = jnp.arange(OWp)
    mats = []
    for di in range(3):
        M = jnp.zeros((Wp, Cin, OWp, Cout), w.dtype)
        for dj in range(3):
            M = M.at[j + dj, :, j, :].add(w4[:, di, dj, :])
        if pool:
            M = M.reshape(Wp * Cin, OWp // 2, 2, Cout).transpose(0, 2, 1, 3)
        mats.append(M.reshape(Wp * Cin, OWp * Cout))
    return mats


def _bconv(x, pieces, brow, pool, OHv, pad_to):
    B, Hp, WC = x.shape
    M = B * Hp - 2
    outs = []
    for lane_lo, ms in pieces:
        K = ms[0].shape[0]
        OWC = ms[0].shape[1]
        F = x[:, :, lane_lo:lane_lo + K].reshape(B * Hp, K)
        acc = jnp.dot(F[0:M], ms[0], preferred_element_type=jnp.float32)
        acc = acc + jnp.dot(F[1:1 + M], ms[1],
                            preferred_element_type=jnp.float32)
        acc = acc + jnp.dot(F[2:2 + M], ms[2],
                            preferred_element_type=jnp.float32)
        o = jnp.maximum(acc + brow[:, :OWC], 0.0).astype(jnp.bfloat16)
        o = jnp.concatenate([o, jnp.zeros((2, OWC), o.dtype)], axis=0)
        o = o.reshape(B, Hp, OWC)
        if pool:
            v = o[:, :OHv, :].reshape(B, OHv // 2, 2, OWC)
            m = jnp.maximum(v[:, :, 0, :], v[:, :, 1, :])
            o = jnp.maximum(m[:, :, :OWC // 2], m[:, :, OWC // 2:])
        outs.append(o)
    o = outs[0] if len(outs) == 1 else jnp.concatenate(outs, axis=2)
    rows = o.shape[1]
    if rows < pad_to:
        o = jnp.concatenate(
            [o, jnp.zeros((B, pad_to - rows, o.shape[2]), o.dtype)], axis=1)
    return o


def _ffnet_kernel(cfg, x_ref, *refs):
    o_ref = refs[-1]
    vals = [r[...] for r in refs[:-1]]
    B = x_ref.shape[0]
    a = x_ref[...]
    idx = 0
    for lane_los, pool, OHv, pad_to in cfg:
        pieces = []
        for lo in lane_los:
            pieces.append((lo, vals[idx:idx + 3]))
            idx += 3
        brow = vals[idx]
        idx += 1
        a = _bconv(a, pieces, brow, pool, OHv, pad_to)
    f1w, f1b, f2w, f2b = vals[idx:idx + 4]
    h = jnp.dot(a.reshape(B, 1600), f1w, preferred_element_type=jnp.float32)
    h = jnp.maximum(h + f1b, 0.0).astype(jnp.bfloat16)
    y = jnp.dot(h, f2w, preferred_element_type=jnp.float32)
    o_ref[...] = y + f2b


@jax.jit
def _run(x_nchw, c1w, c1b, c2w, c2b, c3w, c3b, c4w, c4b,
         f1w, f1b, f2w, f2b):
    N = x_nchw.shape[0]
    B = _B
    while N % B:
        B //= 2
    x = jnp.transpose(x_nchw, (0, 2, 3, 1)).astype(jnp.bfloat16)
    x = x.reshape(N, 32, 32 * 3)

    ws = []
    cfg = []
    for (w, b), (W, splits, pool, pad_to) in zip(
            ((c1w, c1b), (c2w, c2b), (c3w, c3b), (c4w, c4b)), _CONVS):
        Cin, Cout = w.shape[0], b.shape[1]
        for j_lo, j_hi in zip(splits[:-1], splits[1:]):
            ws += _band_mats(w, Cout, j_lo, j_hi, pool)
        ws.append(jnp.tile(b, (1, W - 2)))
        cfg.append((tuple(lo * Cin for lo in splits[:-1]), pool,
                    W - 2, pad_to))
    ws += [f1w, f1b, f2w, f2b]

    def _full(w):
        return pl.BlockSpec(w.shape, lambda i, n=w.ndim: (0,) * n)

    out = pl.pallas_call(
        functools.partial(_ffnet_kernel, tuple(cfg)),
        out_shape=jax.ShapeDtypeStruct((N, 128), jnp.float32),
        grid=(N // B,),
        in_specs=[pl.BlockSpec((B, 32, 32 * 3), lambda i: (i, 0, 0))]
        + [_full(w) for w in ws],
        out_specs=pl.BlockSpec((B, 128), lambda i: (i, 0)),
        compiler_params=pltpu.CompilerParams(
            dimension_semantics=("parallel",)),
    )(x, *ws)
    return out[:, :10]


def kernel(x_nchw, conv1_w, conv1_b, conv2_w, conv2_b, conv3_w, conv3_b,
           conv4_w, conv4_b, fc1_w, fc1_b, fc2_w, fc2_b):
    return _run(x_nchw, conv1_w, conv1_b, conv2_w, conv2_b, conv3_w, conv3_b,
                conv4_w, conv4_b, fc1_w, fc1_b, fc2_w, fc2_b)
